# Initial kernel scaffold; baseline (speedup 1.0000x reference)
import jax
import jax.numpy as jnp
from jax.experimental import pallas as pl


def kernel(boxes, scores, labels):
    raise NotImplementedError("write your pallas kernel here")


# speedup vs baseline: 851.1742x; 851.1742x over previous
import functools

import jax
import jax.numpy as jnp
from jax import lax
from jax.experimental import pallas as pl
from jax.experimental.pallas import tpu as pltpu
from jax.experimental.pallas import tpu_sc as plsc

N = 5000
NP = 5008
NCHUNK = NP // 16
CAP = NP + 16
NW = 16
CPW = 5
HUMAN = 1
MAXK = 15
SCORE_THRESH = 0.2
NMS_THRESH = 0.5
NEG = -1e30
BIGI = 1 << 30

_mesh = plsc.VectorSubcoreMesh(
    core_axis_name="c", subcore_axis_name="s", num_cores=1)


@functools.partial(
    pl.kernel,
    out_type=jax.ShapeDtypeStruct((30, 5), jnp.float32),
    mesh=_mesh,
    scratch_types=[
        pltpu.VMEM((CAP,), jnp.float32),
        pltpu.VMEM((CAP,), jnp.float32),
        pltpu.VMEM((CAP,), jnp.float32),
        pltpu.VMEM((CAP,), jnp.float32),
        pltpu.VMEM((NP,), jnp.float32),
        pltpu.VMEM((NP,), jnp.int32),
        pltpu.VMEM((CAP,), jnp.float32),
        pltpu.VMEM((CAP,), jnp.int32),
        pltpu.VMEM((CAP,), jnp.int32),
        pltpu.VMEM((CAP,), jnp.float32),
        pltpu.VMEM((CAP,), jnp.int32),
        pltpu.VMEM((CAP,), jnp.float32),
        pltpu.VMEM((CAP,), jnp.float32),
        pltpu.VMEM((CAP,), jnp.float32),
        pltpu.VMEM((CAP,), jnp.float32),
        pltpu.VMEM((CPW * 16,), jnp.float32),
        pltpu.VMEM((CPW * 16,), jnp.int32),
        pltpu.VMEM((16,), jnp.float32),
        pltpu.VMEM((16,), jnp.int32),
        pltpu.VMEM_SHARED(((NW + 1) * 16,), jnp.float32),
        pltpu.VMEM_SHARED(((NW + 1) * 16,), jnp.int32),
        pltpu.VMEM(((NW + 1) * 16,), jnp.float32),
        pltpu.VMEM(((NW + 1) * 16,), jnp.int32),
        pltpu.VMEM((30, 5), jnp.float32),
    ],
    compiler_params=pltpu.CompilerParams(needs_layout_passes=False),
)
def _nms_sc(x1h, y1h, x2h, y2h, sch, labh, outh,
            x1v, y1v, x2v, y2v, scv, labv,
            ps, pi, plab, cs, ci, cx1, cy1, cx2, cy2,
            locs, loci, tops, topi, shs, shi, msv, miv, outv):
    wid = lax.axis_index("s")
    iota = lax.iota(jnp.int32, 16)
    lane0 = iota == 0

    def splat_f(x):
        return jnp.full((16,), 0.0, jnp.float32) + x

    def splat_i(x):
        return jnp.full((16,), 0, jnp.int32) + x

    pltpu.sync_copy(x1h, x1v.at[pl.ds(0, NP)])
    pltpu.sync_copy(y1h, y1v.at[pl.ds(0, NP)])
    pltpu.sync_copy(x2h, x2v.at[pl.ds(0, NP)])
    pltpu.sync_copy(y2h, y2v.at[pl.ds(0, NP)])
    pltpu.sync_copy(sch, scv)
    pltpu.sync_copy(labh, labv)

    def mx_body(j, mvc):
        b = j * 16
        mvc = jnp.maximum(mvc, x2v[pl.ds(b, 16)])
        return jnp.maximum(mvc, y2v[pl.ds(b, 16)])
    mvec = lax.fori_loop(0, NCHUNK, mx_body, jnp.zeros((16,), jnp.float32))
    maxc = jnp.max(mvec)

    def p1_body(j, wp):
        b = j * 16
        sc = scv[pl.ds(b, 16)]
        lab = labv[pl.ds(b, 16)]
        m = (sc >= SCORE_THRESH) & ((lab & 15) == wid)
        plsc.store_compressed(ps.at[pl.ds(wp, 16)], sc, mask=m)
        plsc.store_compressed(pi.at[pl.ds(wp, 16)], b + iota, mask=m)
        plsc.store_compressed(plab.at[pl.ds(wp, 16)], lab, mask=m)
        return wp + plsc.all_reduce_population_count(m)[0]
    npool = lax.fori_loop(0, NCHUNK, p1_body, jnp.int32(0))
    npc = (npool + 15) // 16

    def class_body(k, _):
        c = wid + 16 * k

        def cb(j, cp):
            b = j * 16
            lab = plab[pl.ds(b, 16)]
            s = ps[pl.ds(b, 16)]
            ii = pi[pl.ds(b, 16)]
            m = ((b + iota) < npool) & (lab == c)
            plsc.store_compressed(cs.at[pl.ds(cp, 16)], s, mask=m)
            plsc.store_compressed(ci.at[pl.ds(cp, 16)], ii, mask=m)
            return cp + plsc.all_reduce_population_count(m)[0]
        nc = lax.fori_loop(0, npc, cb, jnp.int32(0))
        ncc = (nc + 15) // 16

        offc = c.astype(jnp.float32) * (maxc + 1.0)

        def gb(j, _):
            b = j * 16
            iv = ci[pl.ds(b, 16)]
            mval = (b + iota) < nc
            cx1[pl.ds(b, 16)] = plsc.load_gather(x1v, [iv], mask=mval) + offc
            cy1[pl.ds(b, 16)] = plsc.load_gather(y1v, [iv], mask=mval) + offc
            cx2[pl.ds(b, 16)] = plsc.load_gather(x2v, [iv], mask=mval) + offc
            cy2[pl.ds(b, 16)] = plsc.load_gather(y2v, [iv], mask=mval) + offc
            return 0
        lax.fori_loop(0, ncc, gb, jnp.int32(0))

        def sel(t, carry):
            ks, ki = carry

            def am(j, amc):
                bs, bp = amc
                b = j * 16
                pos = b + iota
                s = jnp.where(pos < nc, cs[pl.ds(b, 16)], NEG)
                upd = s > bs
                return jnp.where(upd, s, bs), jnp.where(upd, pos, bp)
            bs, bp = lax.fori_loop(
                0, ncc, am,
                (jnp.full((16,), NEG, jnp.float32),
                 jnp.full((16,), BIGI, jnp.int32)))
            m = jnp.max(bs)
            found = m >= 0.0
            p = jnp.where(found, jnp.min(jnp.where(bs == m, bp, BIGI)), 0)
            sv = cs[pl.ds(p, 16)][0]
            iv = ci[pl.ds(p, 16)][0]
            hit = found & (iota == t)
            ks = jnp.where(hit, sv, ks)
            ki = jnp.where(hit, iv, ki)

            @pl.when(found)
            def _():
                bx1 = cx1[pl.ds(p, 16)][0]
                by1 = cy1[pl.ds(p, 16)][0]
                bx2 = cx2[pl.ds(p, 16)][0]
                by2 = cy2[pl.ds(p, 16)][0]
                barea = (bx2 - bx1) * (by2 - by1)

                def sup(j, _):
                    b = j * 16
                    X1 = cx1[pl.ds(b, 16)]
                    Y1 = cy1[pl.ds(b, 16)]
                    X2 = cx2[pl.ds(b, 16)]
                    Y2 = cy2[pl.ds(b, 16)]
                    s2 = cs[pl.ds(b, 16)]
                    ix = jnp.maximum(
                        jnp.minimum(bx2, X2) - jnp.maximum(bx1, X1), 0.0)
                    iy = jnp.maximum(
                        jnp.minimum(by2, Y2) - jnp.maximum(by1, Y1), 0.0)
                    inter = ix * iy
                    areas = (X2 - X1) * (Y2 - Y1)
                    union = barea + areas - inter
                    iou = inter / jnp.maximum(union, 1e-9)
                    cs[pl.ds(b, 16)] = jnp.where(iou > NMS_THRESH, NEG, s2)
                    return 0
                lax.fori_loop(0, ncc, sup, jnp.int32(0))
            return ks, ki
        ks, ki = lax.fori_loop(
            0, MAXK, sel,
            (jnp.full((16,), NEG, jnp.float32),
             jnp.zeros((16,), jnp.int32)))
        locs[pl.ds(k * 16, 16)] = ks
        loci[pl.ds(k * 16, 16)] = ki
        return 0
    lax.fori_loop(0, CPW, class_body, jnp.int32(0))

    def lt(t, carry):
        ts, ti = carry

        def am(j, amc):
            bs, bi, bp = amc
            b = j * 16
            s = locs[pl.ds(b, 16)]
            ii = loci[pl.ds(b, 16)]
            hm = (wid == HUMAN) & (j == 0)
            s = jnp.where(hm, NEG, s)
            upd = (s > bs) | ((s == bs) & (ii < bi))
            return (jnp.where(upd, s, bs), jnp.where(upd, ii, bi),
                    jnp.where(upd, b + iota, bp))
        bs, bi, bp = lax.fori_loop(
            0, CPW, am,
            (jnp.full((16,), NEG, jnp.float32),
             jnp.full((16,), BIGI, jnp.int32),
             jnp.full((16,), BIGI, jnp.int32)))
        m = jnp.max(bs)
        found = m >= 0.0
        im = jnp.min(jnp.where(bs == m, bi, BIGI))
        hit = found & (iota == t)
        ts = jnp.where(hit, m, ts)
        ti = jnp.where(hit, im, ti)

        @pl.when(found)
        def _():
            p = jnp.min(jnp.where((bs == m) & (bi == im), bp, BIGI))
            plsc.store_scatter(locs, [splat_i(p)], splat_f(NEG), mask=lane0)
        return ts, ti
    ts, ti = lax.fori_loop(
        0, MAXK, lt,
        (jnp.full((16,), NEG, jnp.float32), jnp.zeros((16,), jnp.int32)))
    tops[...] = ts
    topi[...] = ti

    pltpu.sync_copy(tops, shs.at[pl.ds(wid * 16, 16)])
    pltpu.sync_copy(topi, shi.at[pl.ds(wid * 16, 16)])

    @pl.when(wid == HUMAN)
    def _():
        pltpu.sync_copy(locs.at[pl.ds(0, 16)], shs.at[pl.ds(NW * 16, 16)])
        pltpu.sync_copy(loci.at[pl.ds(0, 16)], shi.at[pl.ds(NW * 16, 16)])

    plsc.subcore_barrier()

    @pl.when(wid == 0)
    def _():
        pltpu.sync_copy(shs, msv)
        pltpu.sync_copy(shi, miv)

        def gt(t, carry):
            osv, oiv = carry

            def am(j, amc):
                bs, bi, bp = amc
                b = j * 16
                s = msv[pl.ds(b, 16)]
                ii = miv[pl.ds(b, 16)]
                upd = (s > bs) | ((s == bs) & (ii < bi))
                return (jnp.where(upd, s, bs), jnp.where(upd, ii, bi),
                        jnp.where(upd, b + iota, bp))
            bs, bi, bp = lax.fori_loop(
                0, NW, am,
                (jnp.full((16,), NEG, jnp.float32),
                 jnp.full((16,), BIGI, jnp.int32),
                 jnp.full((16,), BIGI, jnp.int32)))
            m = jnp.max(bs)
            found = m >= 0.0
            im = jnp.min(jnp.where(bs == m, bi, BIGI))
            hit = found & (iota == t)
            osv = jnp.where(hit, m, osv)
            oiv = jnp.where(hit, im, oiv)

            @pl.when(found)
            def _():
                p = jnp.min(jnp.where((bs == m) & (bi == im), bp, BIGI))
                plsc.store_scatter(msv, [splat_i(p)], splat_f(NEG),
                                   mask=lane0)
            return osv, oiv
        osv, oiv = lax.fori_loop(
            0, MAXK, gt,
            (jnp.full((16,), NEG, jnp.float32),
             jnp.zeros((16,), jnp.int32)))

        hsv = msv[pl.ds(NW * 16, 16)]
        hiv = miv[pl.ds(NW * 16, 16)]

        def emit(svec, ivec, row0):
            valid = svec >= 0.0
            rows = row0 + iota
            rmask = iota < MAXK
            gx1 = plsc.load_gather(x1v, [ivec])
            gy1 = plsc.load_gather(y1v, [ivec])
            gx2 = plsc.load_gather(x2v, [ivec])
            gy2 = plsc.load_gather(y2v, [ivec])
            z = jnp.zeros((16,), jnp.float32)
            plsc.store_scatter(outv, [rows, splat_i(0)],
                               jnp.where(valid, gx1, z), mask=rmask)
            plsc.store_scatter(outv, [rows, splat_i(1)],
                               jnp.where(valid, gy1, z), mask=rmask)
            plsc.store_scatter(outv, [rows, splat_i(2)],
                               jnp.where(valid, gx2, z), mask=rmask)
            plsc.store_scatter(outv, [rows, splat_i(3)],
                               jnp.where(valid, gy2, z), mask=rmask)
            plsc.store_scatter(outv, [rows, splat_i(4)],
                               jnp.where(valid, svec, z), mask=rmask)

        emit(hsv, jnp.where(hsv >= 0.0, hiv, 0), 0)
        emit(osv, oiv, MAXK)

        pltpu.sync_copy(outv, outh)


def kernel(boxes, scores, labels):
    pad = NP - N
    x1 = jnp.pad(boxes[:, 0], (0, pad))
    y1 = jnp.pad(boxes[:, 1], (0, pad))
    x2 = jnp.pad(boxes[:, 2], (0, pad))
    y2 = jnp.pad(boxes[:, 3], (0, pad))
    sc = jnp.pad(scores, (0, pad), constant_values=-1.0)
    lab = jnp.pad(labels.astype(jnp.int32), (0, pad))
    return _nms_sc(x1, y1, x2, y2, sc, lab)


# speedup vs baseline: 958.0516x; 1.1256x over previous
import functools

import jax
import jax.numpy as jnp
from jax import lax
from jax.experimental import pallas as pl
from jax.experimental.pallas import tpu as pltpu
from jax.experimental.pallas import tpu_sc as plsc

N = 5000
SEG = 5008
NCHUNK = SEG // 16
FLAT = 6 * SEG + 16
CAP = SEG + 16
NW = 16
CPW = 5
HUMAN = 1
MAXK = 15
SCORE_THRESH = 0.2
NMS_THRESH = 0.5
NEG = -1e30
BIGI = 1 << 30
OX1, OY1, OX2, OY2, OSC, OLAB = (0, SEG, 2 * SEG, 3 * SEG, 4 * SEG, 5 * SEG)

_mesh = plsc.VectorSubcoreMesh(
    core_axis_name="c", subcore_axis_name="s", num_cores=1)


@functools.partial(
    pl.kernel,
    out_type=jax.ShapeDtypeStruct((30, 5), jnp.float32),
    mesh=_mesh,
    scratch_types=[
        pltpu.VMEM((FLAT,), jnp.float32),
        pltpu.VMEM((CAP,), jnp.int32),
        pltpu.VMEM((CAP,), jnp.float32),
        pltpu.VMEM((CAP,), jnp.int32),
        pltpu.VMEM((CAP,), jnp.float32),
        pltpu.VMEM((CAP,), jnp.float32),
        pltpu.VMEM((CAP,), jnp.float32),
        pltpu.VMEM((CAP,), jnp.float32),
        pltpu.VMEM((CPW * 16,), jnp.float32),
        pltpu.VMEM((CPW * 16,), jnp.int32),
        pltpu.VMEM((16,), jnp.float32),
        pltpu.VMEM((16,), jnp.int32),
        pltpu.VMEM_SHARED(((NW + 1) * 16,), jnp.float32),
        pltpu.VMEM_SHARED(((NW + 1) * 16,), jnp.int32),
        pltpu.VMEM(((NW + 1) * 16,), jnp.float32),
        pltpu.VMEM(((NW + 1) * 16,), jnp.int32),
        pltpu.VMEM((30, 5), jnp.float32),
    ],
    compiler_params=pltpu.CompilerParams(needs_layout_passes=False),
)
def _nms_sc(fh, outh,
            fv, pi, cs, ci, cx1, cy1, cx2, cy2,
            locs, loci, tops, topi, shs, shi, msv, miv, outv):
    wid = lax.axis_index("s")
    iota = lax.iota(jnp.int32, 16)
    lane0 = iota == 0

    def splat_f(x):
        return jnp.full((16,), 0.0, jnp.float32) + x

    def splat_i(x):
        return jnp.full((16,), 0, jnp.int32) + x

    pltpu.sync_copy(fh, fv.at[pl.ds(0, 6 * SEG)])

    def p1_body(j, carry):
        wp, mvc = carry
        b = j * 16
        sc = fv[pl.ds(OSC + b, 16)]
        lab = fv[pl.ds(OLAB + b, 16)].astype(jnp.int32)
        m = (sc >= SCORE_THRESH) & ((lab & 15) == wid)
        plsc.store_compressed(pi.at[pl.ds(wp, 16)], b + iota, mask=m)
        mvc = jnp.maximum(mvc, fv[pl.ds(OX2 + b, 16)])
        mvc = jnp.maximum(mvc, fv[pl.ds(OY2 + b, 16)])
        return wp + plsc.all_reduce_population_count(m)[0], mvc
    npool, mvec = lax.fori_loop(
        0, NCHUNK, p1_body, (jnp.int32(0), jnp.zeros((16,), jnp.float32)))
    maxc = jnp.max(mvec)
    npc = (npool + 15) // 16

    def class_body(k, _):
        c = wid + 16 * k
        cf = c.astype(jnp.float32)

        def cb(j, cp):
            b = j * 16
            ii = pi[pl.ds(b, 16)]
            mv = (b + iota) < npool
            labg = plsc.load_gather(fv, [ii + OLAB], mask=mv)
            m = mv & (labg == cf)
            plsc.store_compressed(ci.at[pl.ds(cp, 16)], ii, mask=m)
            return cp + plsc.all_reduce_population_count(m)[0]
        nc = lax.fori_loop(0, npc, cb, jnp.int32(0))
        ncc = (nc + 15) // 16

        offc = cf * (maxc + 1.0)

        def gb(j, _):
            b = j * 16
            iv = ci[pl.ds(b, 16)]
            mval = (b + iota) < nc
            cs[pl.ds(b, 16)] = plsc.load_gather(fv, [iv + OSC], mask=mval)
            cx1[pl.ds(b, 16)] = plsc.load_gather(fv, [iv], mask=mval) + offc
            cy1[pl.ds(b, 16)] = (
                plsc.load_gather(fv, [iv + OY1], mask=mval) + offc)
            cx2[pl.ds(b, 16)] = (
                plsc.load_gather(fv, [iv + OX2], mask=mval) + offc)
            cy2[pl.ds(b, 16)] = (
                plsc.load_gather(fv, [iv + OY2], mask=mval) + offc)
            return 0
        lax.fori_loop(0, ncc, gb, jnp.int32(0))

        def sel(t, carry):
            ks, ki = carry

            def am(j, amc):
                bs, bp = amc
                b = j * 16
                pos = b + iota
                s = jnp.where(pos < nc, cs[pl.ds(b, 16)], NEG)
                upd = s > bs
                return jnp.where(upd, s, bs), jnp.where(upd, pos, bp)
            bs, bp = lax.fori_loop(
                0, ncc, am,
                (jnp.full((16,), NEG, jnp.float32),
                 jnp.full((16,), BIGI, jnp.int32)))
            m = jnp.max(bs)
            found = m >= 0.0
            p = jnp.where(found, jnp.min(jnp.where(bs == m, bp, BIGI)), 0)
            sv = cs[pl.ds(p, 16)][0]
            iv = ci[pl.ds(p, 16)][0]
            hit = found & (iota == t)
            ks = jnp.where(hit, sv, ks)
            ki = jnp.where(hit, iv, ki)

            @pl.when(found)
            def _():
                bx1 = cx1[pl.ds(p, 16)][0]
                by1 = cy1[pl.ds(p, 16)][0]
                bx2 = cx2[pl.ds(p, 16)][0]
                by2 = cy2[pl.ds(p, 16)][0]
                barea = (bx2 - bx1) * (by2 - by1)

                def sup(j, _):
                    b = j * 16
                    X1 = cx1[pl.ds(b, 16)]
                    Y1 = cy1[pl.ds(b, 16)]
                    X2 = cx2[pl.ds(b, 16)]
                    Y2 = cy2[pl.ds(b, 16)]
                    s2 = cs[pl.ds(b, 16)]
                    ix = jnp.maximum(
                        jnp.minimum(bx2, X2) - jnp.maximum(bx1, X1), 0.0)
                    iy = jnp.maximum(
                        jnp.minimum(by2, Y2) - jnp.maximum(by1, Y1), 0.0)
                    inter = ix * iy
                    areas = (X2 - X1) * (Y2 - Y1)
                    union = barea + areas - inter
                    iou = inter / jnp.maximum(union, 1e-9)
                    cs[pl.ds(b, 16)] = jnp.where(iou > NMS_THRESH, NEG, s2)
                    return 0
                lax.fori_loop(0, ncc, sup, jnp.int32(0))
            return ks, ki
        ks, ki = lax.fori_loop(
            0, MAXK, sel,
            (jnp.full((16,), NEG, jnp.float32),
             jnp.zeros((16,), jnp.int32)))
        locs[pl.ds(k * 16, 16)] = ks
        loci[pl.ds(k * 16, 16)] = ki
        return 0
    lax.fori_loop(0, CPW, class_body, jnp.int32(0))

    def lt(t, carry):
        ts, ti = carry

        def am(j, amc):
            bs, bi, bp = amc
            b = j * 16
            s = locs[pl.ds(b, 16)]
            ii = loci[pl.ds(b, 16)]
            hm = (wid == HUMAN) & (j == 0)
            s = jnp.where(hm, NEG, s)
            upd = (s > bs) | ((s == bs) & (ii < bi))
            return (jnp.where(upd, s, bs), jnp.where(upd, ii, bi),
                    jnp.where(upd, b + iota, bp))
        bs, bi, bp = lax.fori_loop(
            0, CPW, am,
            (jnp.full((16,), NEG, jnp.float32),
             jnp.full((16,), BIGI, jnp.int32),
             jnp.full((16,), BIGI, jnp.int32)))
        m = jnp.max(bs)
        found = m >= 0.0
        im = jnp.min(jnp.where(bs == m, bi, BIGI))
        hit = found & (iota == t)
        ts = jnp.where(hit, m, ts)
        ti = jnp.where(hit, im, ti)

        @pl.when(found)
        def _():
            p = jnp.min(jnp.where((bs == m) & (bi == im), bp, BIGI))
            plsc.store_scatter(locs, [splat_i(p)], splat_f(NEG), mask=lane0)
        return ts, ti
    ts, ti = lax.fori_loop(
        0, MAXK, lt,
        (jnp.full((16,), NEG, jnp.float32), jnp.zeros((16,), jnp.int32)))
    tops[...] = ts
    topi[...] = ti

    pltpu.sync_copy(tops, shs.at[pl.ds(wid * 16, 16)])
    pltpu.sync_copy(topi, shi.at[pl.ds(wid * 16, 16)])

    @pl.when(wid == HUMAN)
    def _():
        pltpu.sync_copy(locs.at[pl.ds(0, 16)], shs.at[pl.ds(NW * 16, 16)])
        pltpu.sync_copy(loci.at[pl.ds(0, 16)], shi.at[pl.ds(NW * 16, 16)])

    plsc.subcore_barrier()

    @pl.when(wid == 0)
    def _():
        pltpu.sync_copy(shs, msv)
        pltpu.sync_copy(shi, miv)

        def gt(t, carry):
            osv, oiv = carry

            def am(j, amc):
                bs, bi, bp = amc
                b = j * 16
                s = msv[pl.ds(b, 16)]
                ii = miv[pl.ds(b, 16)]
                upd = (s > bs) | ((s == bs) & (ii < bi))
                return (jnp.where(upd, s, bs), jnp.where(upd, ii, bi),
                        jnp.where(upd, b + iota, bp))
            bs, bi, bp = lax.fori_loop(
                0, NW, am,
                (jnp.full((16,), NEG, jnp.float32),
                 jnp.full((16,), BIGI, jnp.int32),
                 jnp.full((16,), BIGI, jnp.int32)))
            m = jnp.max(bs)
            found = m >= 0.0
            im = jnp.min(jnp.where(bs == m, bi, BIGI))
            hit = found & (iota == t)
            osv = jnp.where(hit, m, osv)
            oiv = jnp.where(hit, im, oiv)

            @pl.when(found)
            def _():
                p = jnp.min(jnp.where((bs == m) & (bi == im), bp, BIGI))
                plsc.store_scatter(msv, [splat_i(p)], splat_f(NEG),
                                   mask=lane0)
            return osv, oiv
        osv, oiv = lax.fori_loop(
            0, MAXK, gt,
            (jnp.full((16,), NEG, jnp.float32),
             jnp.zeros((16,), jnp.int32)))

        hsv = msv[pl.ds(NW * 16, 16)]
        hiv = miv[pl.ds(NW * 16, 16)]

        def emit(svec, ivec, row0):
            valid = svec >= 0.0
            rows = row0 + iota
            rmask = iota < MAXK
            gx1 = plsc.load_gather(fv, [ivec])
            gy1 = plsc.load_gather(fv, [ivec + OY1])
            gx2 = plsc.load_gather(fv, [ivec + OX2])
            gy2 = plsc.load_gather(fv, [ivec + OY2])
            z = jnp.zeros((16,), jnp.float32)
            plsc.store_scatter(outv, [rows, splat_i(0)],
                               jnp.where(valid, gx1, z), mask=rmask)
            plsc.store_scatter(outv, [rows, splat_i(1)],
                               jnp.where(valid, gy1, z), mask=rmask)
            plsc.store_scatter(outv, [rows, splat_i(2)],
                               jnp.where(valid, gx2, z), mask=rmask)
            plsc.store_scatter(outv, [rows, splat_i(3)],
                               jnp.where(valid, gy2, z), mask=rmask)
            plsc.store_scatter(outv, [rows, splat_i(4)],
                               jnp.where(valid, svec, z), mask=rmask)

        emit(hsv, jnp.where(hsv >= 0.0, hiv, 0), 0)
        emit(osv, oiv, MAXK)

        pltpu.sync_copy(outv, outh)


def kernel(boxes, scores, labels):
    pad = SEG - N
    flat = jnp.concatenate([
        jnp.pad(boxes[:, 0], (0, pad)),
        jnp.pad(boxes[:, 1], (0, pad)),
        jnp.pad(boxes[:, 2], (0, pad)),
        jnp.pad(boxes[:, 3], (0, pad)),
        jnp.pad(scores, (0, pad), constant_values=-1.0),
        jnp.pad(labels.astype(jnp.float32), (0, pad)),
    ])
    return _nms_sc(flat)


# speedup vs baseline: 979.8659x; 1.0228x over previous
import functools

import jax
import jax.numpy as jnp
from jax import lax
from jax.experimental import pallas as pl
from jax.experimental.pallas import tpu as pltpu
from jax.experimental.pallas import tpu_sc as plsc

N = 5000
SEG = 5008
NCHUNK = SEG // 16
FLAT = 6 * SEG + 16
CAP = SEG + 16
NW = 16
CPW = 5
HUMAN = 1
MAXK = 15
SCORE_THRESH = 0.2
NMS_THRESH = 0.5
NEG = -1e30
BIGI = 1 << 30
OX1, OY1, OX2, OY2, OSC, OLAB = (0, SEG, 2 * SEG, 3 * SEG, 4 * SEG, 5 * SEG)

_mesh = plsc.VectorSubcoreMesh(
    core_axis_name="c", subcore_axis_name="s", num_cores=1)


@functools.partial(
    pl.kernel,
    out_type=jax.ShapeDtypeStruct((30, 5), jnp.float32),
    mesh=_mesh,
    scratch_types=[
        pltpu.VMEM((FLAT,), jnp.float32),
        pltpu.VMEM((CAP,), jnp.int32),
        pltpu.VMEM((CAP,), jnp.float32),
        pltpu.VMEM((CAP,), jnp.int32),
        pltpu.VMEM((CAP,), jnp.float32),
        pltpu.VMEM((CAP,), jnp.float32),
        pltpu.VMEM((CAP,), jnp.float32),
        pltpu.VMEM((CAP,), jnp.float32),
        pltpu.VMEM((CPW * 16,), jnp.float32),
        pltpu.VMEM((CPW * 16,), jnp.int32),
        pltpu.VMEM((16,), jnp.float32),
        pltpu.VMEM((16,), jnp.int32),
        pltpu.VMEM_SHARED(((NW + 1) * 16,), jnp.float32),
        pltpu.VMEM_SHARED(((NW + 1) * 16,), jnp.int32),
        pltpu.VMEM(((NW + 1) * 16,), jnp.float32),
        pltpu.VMEM(((NW + 1) * 16,), jnp.int32),
        pltpu.VMEM((30, 5), jnp.float32),
    ],
    compiler_params=pltpu.CompilerParams(needs_layout_passes=False),
)
def _nms_sc(fh, outh,
            fv, pi, cs, ci, cx1, cy1, cx2, cy2,
            locs, loci, tops, topi, shs, shi, msv, miv, outv):
    wid = lax.axis_index("s")
    iota = lax.iota(jnp.int32, 16)
    lane0 = iota == 0

    def splat_f(x):
        return jnp.full((16,), 0.0, jnp.float32) + x

    def splat_i(x):
        return jnp.full((16,), 0, jnp.int32) + x

    pltpu.sync_copy(fh, fv.at[pl.ds(0, 6 * SEG)])

    def p1_body(j, carry):
        wp, mvc = carry
        for u in range(2):
            b = (2 * j + u) * 16
            sc = fv[pl.ds(OSC + b, 16)]
            lab = fv[pl.ds(OLAB + b, 16)].astype(jnp.int32)
            m = (sc >= SCORE_THRESH) & ((lab & 15) == wid)
            plsc.store_compressed(pi.at[pl.ds(wp, 16)], b + iota, mask=m)
            wp = wp + plsc.all_reduce_population_count(m)[0]
            mvc = jnp.maximum(mvc, fv[pl.ds(OX2 + b, 16)])
            mvc = jnp.maximum(mvc, fv[pl.ds(OY2 + b, 16)])
        return wp, mvc
    npool, mvec = lax.fori_loop(
        0, NCHUNK // 2, p1_body, (jnp.int32(0), jnp.zeros((16,), jnp.float32)))
    _b = (NCHUNK - 1) * 16
    _sc = fv[pl.ds(OSC + _b, 16)]
    _lab = fv[pl.ds(OLAB + _b, 16)].astype(jnp.int32)
    _m = (_sc >= SCORE_THRESH) & ((_lab & 15) == wid)
    plsc.store_compressed(pi.at[pl.ds(npool, 16)], _b + iota, mask=_m)
    npool = npool + plsc.all_reduce_population_count(_m)[0]
    mvec = jnp.maximum(mvec, fv[pl.ds(OX2 + _b, 16)])
    mvec = jnp.maximum(mvec, fv[pl.ds(OY2 + _b, 16)])
    maxc = jnp.max(mvec)
    npc = (npool + 15) // 16

    def class_body(k, _):
        c = wid + 16 * k
        cf = c.astype(jnp.float32)

        def cb(j, cp):
            b = j * 16
            ii = pi[pl.ds(b, 16)]
            mv = (b + iota) < npool
            labg = plsc.load_gather(fv, [ii + OLAB], mask=mv)
            m = mv & (labg == cf)
            plsc.store_compressed(ci.at[pl.ds(cp, 16)], ii, mask=m)
            return cp + plsc.all_reduce_population_count(m)[0]
        nc = lax.fori_loop(0, npc, cb, jnp.int32(0))
        ncc = (nc + 15) // 16

        offc = cf * (maxc + 1.0)

        def gb(j, carry):
            bs, bp = carry
            b = j * 16
            iv = ci[pl.ds(b, 16)]
            mval = (b + iota) < nc
            sg = plsc.load_gather(fv, [iv + OSC], mask=mval)
            cs[pl.ds(b, 16)] = sg
            cx1[pl.ds(b, 16)] = plsc.load_gather(fv, [iv], mask=mval) + offc
            cy1[pl.ds(b, 16)] = (
                plsc.load_gather(fv, [iv + OY1], mask=mval) + offc)
            cx2[pl.ds(b, 16)] = (
                plsc.load_gather(fv, [iv + OX2], mask=mval) + offc)
            cy2[pl.ds(b, 16)] = (
                plsc.load_gather(fv, [iv + OY2], mask=mval) + offc)
            s = jnp.where(mval, sg, NEG)
            upd = s > bs
            return jnp.where(upd, s, bs), jnp.where(upd, b + iota, bp)
        bs0, bp0 = lax.fori_loop(
            0, ncc, gb,
            (jnp.full((16,), NEG, jnp.float32),
             jnp.full((16,), BIGI, jnp.int32)))

        def sel(t, carry):
            ks, ki, bs, bp = carry
            m = jnp.max(bs)
            found = m >= 0.0
            p = jnp.where(found, jnp.min(jnp.where(bs == m, bp, BIGI)), 0)
            sv = cs[pl.ds(p, 16)][0]
            iv = ci[pl.ds(p, 16)][0]
            hit = found & (iota == t)
            ks = jnp.where(hit, sv, ks)
            ki = jnp.where(hit, iv, ki)

            def do_sweep():
                bx1 = cx1[pl.ds(p, 16)][0]
                by1 = cy1[pl.ds(p, 16)][0]
                bx2 = cx2[pl.ds(p, 16)][0]
                by2 = cy2[pl.ds(p, 16)][0]
                barea = (bx2 - bx1) * (by2 - by1)

                def sup(j, amc):
                    nbs, nbp = amc
                    b = j * 16
                    X1 = cx1[pl.ds(b, 16)]
                    Y1 = cy1[pl.ds(b, 16)]
                    X2 = cx2[pl.ds(b, 16)]
                    Y2 = cy2[pl.ds(b, 16)]
                    s2 = cs[pl.ds(b, 16)]
                    ix = jnp.maximum(
                        jnp.minimum(bx2, X2) - jnp.maximum(bx1, X1), 0.0)
                    iy = jnp.maximum(
                        jnp.minimum(by2, Y2) - jnp.maximum(by1, Y1), 0.0)
                    inter = ix * iy
                    areas = (X2 - X1) * (Y2 - Y1)
                    union = barea + areas - inter
                    iou = inter / jnp.maximum(union, 1e-9)
                    s_new = jnp.where(iou > NMS_THRESH, NEG, s2)
                    cs[pl.ds(b, 16)] = s_new
                    s = jnp.where((b + iota) < nc, s_new, NEG)
                    upd = s > nbs
                    return (jnp.where(upd, s, nbs),
                            jnp.where(upd, b + iota, nbp))
                return lax.fori_loop(
                    0, ncc, sup,
                    (jnp.full((16,), NEG, jnp.float32),
                     jnp.full((16,), BIGI, jnp.int32)))

            bs, bp = lax.cond(
                found, do_sweep,
                lambda: (jnp.full((16,), NEG, jnp.float32),
                         jnp.full((16,), BIGI, jnp.int32)))
            return ks, ki, bs, bp
        ks, ki, _, _ = lax.fori_loop(
            0, MAXK, sel,
            (jnp.full((16,), NEG, jnp.float32),
             jnp.zeros((16,), jnp.int32), bs0, bp0))
        locs[pl.ds(k * 16, 16)] = ks
        loci[pl.ds(k * 16, 16)] = ki
        return 0
    lax.fori_loop(0, CPW, class_body, jnp.int32(0))

    def lt(t, carry):
        ts, ti = carry

        def am(j, amc):
            bs, bi, bp = amc
            b = j * 16
            s = locs[pl.ds(b, 16)]
            ii = loci[pl.ds(b, 16)]
            hm = (wid == HUMAN) & (j == 0)
            s = jnp.where(hm, NEG, s)
            upd = (s > bs) | ((s == bs) & (ii < bi))
            return (jnp.where(upd, s, bs), jnp.where(upd, ii, bi),
                    jnp.where(upd, b + iota, bp))
        bs, bi, bp = lax.fori_loop(
            0, CPW, am,
            (jnp.full((16,), NEG, jnp.float32),
             jnp.full((16,), BIGI, jnp.int32),
             jnp.full((16,), BIGI, jnp.int32)))
        m = jnp.max(bs)
        found = m >= 0.0
        im = jnp.min(jnp.where(bs == m, bi, BIGI))
        hit = found & (iota == t)
        ts = jnp.where(hit, m, ts)
        ti = jnp.where(hit, im, ti)

        @pl.when(found)
        def _():
            p = jnp.min(jnp.where((bs == m) & (bi == im), bp, BIGI))
            plsc.store_scatter(locs, [splat_i(p)], splat_f(NEG), mask=lane0)
        return ts, ti
    ts, ti = lax.fori_loop(
        0, MAXK, lt,
        (jnp.full((16,), NEG, jnp.float32), jnp.zeros((16,), jnp.int32)))
    tops[...] = ts
    topi[...] = ti

    pltpu.sync_copy(tops, shs.at[pl.ds(wid * 16, 16)])
    pltpu.sync_copy(topi, shi.at[pl.ds(wid * 16, 16)])

    @pl.when(wid == HUMAN)
    def _():
        pltpu.sync_copy(locs.at[pl.ds(0, 16)], shs.at[pl.ds(NW * 16, 16)])
        pltpu.sync_copy(loci.at[pl.ds(0, 16)], shi.at[pl.ds(NW * 16, 16)])

    plsc.subcore_barrier()

    @pl.when(wid == 0)
    def _():
        pltpu.sync_copy(shs, msv)
        pltpu.sync_copy(shi, miv)

        def gt(t, carry):
            osv, oiv = carry

            def am(j, amc):
                bs, bi, bp = amc
                b = j * 16
                s = msv[pl.ds(b, 16)]
                ii = miv[pl.ds(b, 16)]
                upd = (s > bs) | ((s == bs) & (ii < bi))
                return (jnp.where(upd, s, bs), jnp.where(upd, ii, bi),
                        jnp.where(upd, b + iota, bp))
            bs, bi, bp = lax.fori_loop(
                0, NW, am,
                (jnp.full((16,), NEG, jnp.float32),
                 jnp.full((16,), BIGI, jnp.int32),
                 jnp.full((16,), BIGI, jnp.int32)))
            m = jnp.max(bs)
            found = m >= 0.0
            im = jnp.min(jnp.where(bs == m, bi, BIGI))
            hit = found & (iota == t)
            osv = jnp.where(hit, m, osv)
            oiv = jnp.where(hit, im, oiv)

            @pl.when(found)
            def _():
                p = jnp.min(jnp.where((bs == m) & (bi == im), bp, BIGI))
                plsc.store_scatter(msv, [splat_i(p)], splat_f(NEG),
                                   mask=lane0)
            return osv, oiv
        osv, oiv = lax.fori_loop(
            0, MAXK, gt,
            (jnp.full((16,), NEG, jnp.float32),
             jnp.zeros((16,), jnp.int32)))

        hsv = msv[pl.ds(NW * 16, 16)]
        hiv = miv[pl.ds(NW * 16, 16)]

        def emit(svec, ivec, row0):
            valid = svec >= 0.0
            rows = row0 + iota
            rmask = iota < MAXK
            gx1 = plsc.load_gather(fv, [ivec])
            gy1 = plsc.load_gather(fv, [ivec + OY1])
            gx2 = plsc.load_gather(fv, [ivec + OX2])
            gy2 = plsc.load_gather(fv, [ivec + OY2])
            z = jnp.zeros((16,), jnp.float32)
            plsc.store_scatter(outv, [rows, splat_i(0)],
                               jnp.where(valid, gx1, z), mask=rmask)
            plsc.store_scatter(outv, [rows, splat_i(1)],
                               jnp.where(valid, gy1, z), mask=rmask)
            plsc.store_scatter(outv, [rows, splat_i(2)],
                               jnp.where(valid, gx2, z), mask=rmask)
            plsc.store_scatter(outv, [rows, splat_i(3)],
                               jnp.where(valid, gy2, z), mask=rmask)
            plsc.store_scatter(outv, [rows, splat_i(4)],
                               jnp.where(valid, svec, z), mask=rmask)

        emit(hsv, jnp.where(hsv >= 0.0, hiv, 0), 0)
        emit(osv, oiv, MAXK)

        pltpu.sync_copy(outv, outh)


def kernel(boxes, scores, labels):
    pad = SEG - N
    flat = jnp.concatenate([
        jnp.pad(boxes[:, 0], (0, pad)),
        jnp.pad(boxes[:, 1], (0, pad)),
        jnp.pad(boxes[:, 2], (0, pad)),
        jnp.pad(boxes[:, 3], (0, pad)),
        jnp.pad(scores, (0, pad), constant_values=-1.0),
        jnp.pad(labels.astype(jnp.float32), (0, pad)),
    ])
    return _nms_sc(flat)
